# baseline (speedup 1.0000x reference)
import jax
import jax.numpy as jnp
from jax.experimental import pallas as pl
from jax.experimental.pallas import tpu as pltpu

_OFFSET = 2
_MAX_POS = 2048
_D_MODEL = 1024
_SIZES = (128, 896, 896, 128)
_CH = len(_SIZES)
_STARTS = tuple(sum(_SIZES[:c]) for c in range(_CH))
_BUF_ROWS = max(_SIZES) + 8


def _shift_copy_kernel(in_hbm, out_hbm, bufs, obufs, insems, tailsem, outsems):
    reads = []
    for c in range(_CH):
        rows = _SIZES[c] + 8 if c < _CH - 1 else _SIZES[c]
        reads.append(
            pltpu.make_async_copy(
                in_hbm.at[pl.ds(_STARTS[c], rows), :],
                bufs.at[c, pl.ds(0, rows), :],
                insems.at[c],
            )
        )
    tail = pltpu.make_async_copy(
        in_hbm.at[pl.ds(_MAX_POS, _OFFSET), :],
        bufs.at[_CH - 1, pl.ds(_SIZES[-1], _OFFSET), :],
        tailsem,
    )
    for r in reads:
        r.start()
    tail.start()

    writes = []
    for c in range(_CH):
        reads[c].wait()
        if c == _CH - 1:
            tail.wait()
        obufs[c, pl.ds(0, _SIZES[c]), :] = bufs[c, pl.ds(_OFFSET, _SIZES[c]), :]
        w = pltpu.make_async_copy(
            obufs.at[c, pl.ds(0, _SIZES[c]), :],
            out_hbm.at[pl.ds(_STARTS[c], _SIZES[c]), :],
            outsems.at[c],
        )
        w.start()
        writes.append(w)
    for w in writes:
        w.wait()


def kernel(position_weights):
    return pl.pallas_call(
        _shift_copy_kernel,
        in_specs=[pl.BlockSpec(memory_space=pl.ANY)],
        out_specs=pl.BlockSpec(memory_space=pl.ANY),
        scratch_shapes=[
            pltpu.VMEM((_CH, _BUF_ROWS, _D_MODEL), jnp.float32),
            pltpu.VMEM((_CH, max(_SIZES), _D_MODEL), jnp.float32),
            pltpu.SemaphoreType.DMA((_CH,)),
            pltpu.SemaphoreType.DMA,
            pltpu.SemaphoreType.DMA((_CH,)),
        ],
        out_shape=jax.ShapeDtypeStruct((_MAX_POS, _D_MODEL), jnp.float32),
    )(position_weights)


# speedup vs baseline: 1.0280x; 1.0280x over previous
import jax
import jax.numpy as jnp
from jax.experimental import pallas as pl
from jax.experimental.pallas import tpu as pltpu

_OFFSET = 2
_MAX_POS = 2048
_D_MODEL = 1024
_SIZES = (128, 256, 640, 640, 256, 128)
_CH = len(_SIZES)
_STARTS = tuple(sum(_SIZES[:c]) for c in range(_CH))
_BUF_ROWS = max(_SIZES) + 8


def _shift_copy_kernel(in_hbm, out_hbm, bufs, obufs, insems, tailsem, outsems):
    reads = []
    for c in range(_CH):
        rows = _SIZES[c] + 8 if c < _CH - 1 else _SIZES[c]
        reads.append(
            pltpu.make_async_copy(
                in_hbm.at[pl.ds(_STARTS[c], rows), :],
                bufs.at[c, pl.ds(0, rows), :],
                insems.at[c],
            )
        )
    tail = pltpu.make_async_copy(
        in_hbm.at[pl.ds(_MAX_POS, _OFFSET), :],
        bufs.at[_CH - 1, pl.ds(_SIZES[-1], _OFFSET), :],
        tailsem,
    )
    for r in reads:
        r.start()
    tail.start()

    writes = []
    for c in range(_CH):
        reads[c].wait()
        if c == _CH - 1:
            tail.wait()
        obufs[c, pl.ds(0, _SIZES[c]), :] = bufs[c, pl.ds(_OFFSET, _SIZES[c]), :]
        w = pltpu.make_async_copy(
            obufs.at[c, pl.ds(0, _SIZES[c]), :],
            out_hbm.at[pl.ds(_STARTS[c], _SIZES[c]), :],
            outsems.at[c],
        )
        w.start()
        writes.append(w)
    for w in writes:
        w.wait()


def kernel(position_weights):
    return pl.pallas_call(
        _shift_copy_kernel,
        in_specs=[pl.BlockSpec(memory_space=pl.ANY)],
        out_specs=pl.BlockSpec(memory_space=pl.ANY),
        scratch_shapes=[
            pltpu.VMEM((_CH, _BUF_ROWS, _D_MODEL), jnp.float32),
            pltpu.VMEM((_CH, max(_SIZES), _D_MODEL), jnp.float32),
            pltpu.SemaphoreType.DMA((_CH,)),
            pltpu.SemaphoreType.DMA,
            pltpu.SemaphoreType.DMA((_CH,)),
        ],
        out_shape=jax.ShapeDtypeStruct((_MAX_POS, _D_MODEL), jnp.float32),
    )(position_weights)
